# baseline (speedup 1.0000x reference)
import functools

import jax
import jax.numpy as jnp
from jax.experimental import pallas as pl
from jax.experimental.pallas import tpu as pltpu

_LANE = 128
_SUB = 8


def _ceil_to(x, m):
    return ((x + m - 1) // m) * m


def _block_body(g_ref, h0_ref, w_ref, b_ref, out_ref, x_ref, *, gk, nt,
                lanes, rows):
    @pl.when(pl.program_id(0) == 0)
    def _seed():
        x_ref[:, gk:] = h0_ref[...].astype(x_ref.dtype)

    for s in range(nt):
        for j in range(lanes):
            r = pl.ds(j * rows, rows)
            x_ref[r, :gk] = g_ref[s, r].astype(x_ref.dtype)
            h = jnp.maximum(
                jnp.dot(x_ref[r, :], w_ref[...],
                        preferred_element_type=jnp.float32) + b_ref[...],
                0.0,
            )
            x_ref[r, gk:] = h.astype(x_ref.dtype)
            out_ref[s, r] = h.astype(out_ref.dtype)


def kernel(w_cat, b_cat, g_seq, hidden0):
    T, B, input_size = g_seq.shape
    hidden_size = hidden0.shape[1]
    h_pad = w_cat.shape[1]
    in_pad = w_cat.shape[0] - h_pad
    k_pad = in_pad + h_pad
    b_pad = _ceil_to(B, _SUB)

    nt = 1
    while nt < 8 and T % (nt * 2) == 0:
        nt *= 2

    lanes = 1
    while lanes < 2 and b_pad % (lanes * 2 * _SUB) == 0:
        lanes *= 2
    rows = b_pad // lanes

    g_p = g_seq.astype(jnp.float32)
    if (b_pad, in_pad) != (B, input_size):
        g_p = jnp.zeros((T, b_pad, in_pad), jnp.float32).at[
            :, :B, :input_size].set(g_p)
    h0_p = hidden0.astype(jnp.float32)
    if (b_pad, h_pad) != (B, hidden_size):
        h0_p = jnp.zeros((b_pad, h_pad), jnp.float32).at[
            :B, :hidden_size].set(h0_p)

    body = functools.partial(_block_body, gk=in_pad, nt=nt, lanes=lanes,
                             rows=rows)

    h_seq = pl.pallas_call(
        body,
        out_shape=jax.ShapeDtypeStruct((T, b_pad, h_pad), jnp.float32),
        grid=(T // nt,),
        in_specs=[
            pl.BlockSpec((nt, b_pad, in_pad), lambda i: (i, 0, 0)),
            pl.BlockSpec((b_pad, h_pad), lambda i: (0, 0)),
            pl.BlockSpec((k_pad, h_pad), lambda i: (0, 0)),
            pl.BlockSpec((1, h_pad), lambda i: (0, 0)),
        ],
        out_specs=pl.BlockSpec((nt, b_pad, h_pad), lambda i: (i, 0, 0)),
        scratch_shapes=[pltpu.VMEM((b_pad, k_pad), jnp.bfloat16)],
        compiler_params=pltpu.CompilerParams(
            dimension_semantics=("arbitrary",)),
    )(g_p, h0_p, w_cat.astype(jnp.bfloat16), b_cat.astype(jnp.float32))

    if (b_pad, h_pad) != (B, hidden_size):
        h_seq = h_seq[:, :B, :hidden_size]
    return h_seq


# speedup vs baseline: 1.0931x; 1.0931x over previous
import functools

import jax
import jax.numpy as jnp
from jax.experimental import pallas as pl
from jax.experimental.pallas import tpu as pltpu

_SUB = 8


def _ceil_to(x, m):
    return ((x + m - 1) // m) * m


def _block_body(g_ref, h0_ref, w_ref, b_ref, out_ref, x_ref, *, gk, nt):
    @pl.when(pl.program_id(0) == 0)
    def _seed():
        x_ref[:, gk:] = h0_ref[...]

    for s in range(nt):
        x_ref[:, :gk] = g_ref[s]
        h = jnp.maximum(
            jnp.dot(x_ref[...], w_ref[...],
                    preferred_element_type=jnp.float32) + b_ref[...],
            0.0,
        )
        x_ref[:, gk:] = h
        out_ref[s] = h


def kernel(w_cat, b_cat, g_seq, hidden0):
    T, B, input_size = g_seq.shape
    hidden_size = hidden0.shape[1]
    h_pad = w_cat.shape[1]
    in_pad = w_cat.shape[0] - h_pad
    k_pad = in_pad + h_pad
    b_pad = _ceil_to(B, _SUB)

    nt = 1
    while nt < 16 and T % (nt * 2) == 0:
        nt *= 2

    g_p = g_seq.astype(jnp.float32)
    if (b_pad, in_pad) != (B, input_size):
        g_p = jnp.zeros((T, b_pad, in_pad), jnp.float32).at[
            :, :B, :input_size].set(g_p)
    h0_p = hidden0.astype(jnp.float32)
    if (b_pad, h_pad) != (B, hidden_size):
        h0_p = jnp.zeros((b_pad, h_pad), jnp.float32).at[
            :B, :hidden_size].set(h0_p)

    body = functools.partial(_block_body, gk=in_pad, nt=nt)

    h_seq = pl.pallas_call(
        body,
        out_shape=jax.ShapeDtypeStruct((T, b_pad, h_pad), jnp.float32),
        grid=(T // nt,),
        in_specs=[
            pl.BlockSpec((nt, b_pad, in_pad), lambda i: (i, 0, 0)),
            pl.BlockSpec((b_pad, h_pad), lambda i: (0, 0)),
            pl.BlockSpec((k_pad, h_pad), lambda i: (0, 0)),
            pl.BlockSpec((1, h_pad), lambda i: (0, 0)),
        ],
        out_specs=pl.BlockSpec((nt, b_pad, h_pad), lambda i: (i, 0, 0)),
        scratch_shapes=[pltpu.VMEM((b_pad, k_pad), jnp.float32)],
        compiler_params=pltpu.CompilerParams(
            dimension_semantics=("arbitrary",)),
    )(g_p, h0_p, w_cat.astype(jnp.float32), b_cat.astype(jnp.float32))

    if (b_pad, h_pad) != (B, hidden_size):
        h_seq = h_seq[:, :B, :hidden_size]
    return h_seq


# speedup vs baseline: 1.1347x; 1.0380x over previous
import functools

import jax
import jax.numpy as jnp
from jax.experimental import pallas as pl
from jax.experimental.pallas import tpu as pltpu

_SUB = 8


def _ceil_to(x, m):
    return ((x + m - 1) // m) * m


def _block_body(g_ref, h0_ref, w_ref, b_ref, out_hbm, x_ref, obuf, osem,
                *, gk, nt, chunk):
    i = pl.program_id(0)
    n_chunks = nt // chunk
    n_slots = 2 if n_chunks % 2 == 0 else 1

    @pl.when(i == 0)
    def _seed():
        x_ref[:, gk:] = h0_ref[...]

    for c in range(n_chunks):
        slot = c % n_slots
        chunk_idx = i * n_chunks + c

        @pl.when(chunk_idx >= n_slots)
        def _reclaim():
            pltpu.make_async_copy(
                obuf.at[slot], out_hbm.at[pl.ds(0, chunk)], osem.at[slot]
            ).wait()

        for s in range(c * chunk, (c + 1) * chunk):
            x_ref[:, :gk] = g_ref[s]
            h = jnp.maximum(
                jnp.dot(x_ref[...], w_ref[...],
                        preferred_element_type=jnp.float32) + b_ref[...],
                0.0,
            )
            x_ref[:, gk:] = h
            obuf[slot, s - c * chunk] = h

        pltpu.make_async_copy(
            obuf.at[slot],
            out_hbm.at[pl.ds(chunk_idx * chunk, chunk)],
            osem.at[slot],
        ).start()

    @pl.when(i == pl.num_programs(0) - 1)
    def _flush():
        for slot in range(n_slots):
            pltpu.make_async_copy(
                obuf.at[slot], out_hbm.at[pl.ds(0, chunk)], osem.at[slot]
            ).wait()


def kernel(w_cat, b_cat, g_seq, hidden0):
    T, B, input_size = g_seq.shape
    hidden_size = hidden0.shape[1]
    h_pad = w_cat.shape[1]
    in_pad = w_cat.shape[0] - h_pad
    k_pad = in_pad + h_pad
    b_pad = _ceil_to(B, _SUB)

    nt = 1
    while nt < 16 and T % (nt * 2) == 0:
        nt *= 2
    chunk = nt
    if nt % 8 == 0:
        chunk = nt // 4
    elif nt % 2 == 0:
        chunk = nt // 2

    g_p = g_seq.astype(jnp.float32)
    if (b_pad, in_pad) != (B, input_size):
        g_p = jnp.zeros((T, b_pad, in_pad), jnp.float32).at[
            :, :B, :input_size].set(g_p)
    h0_p = hidden0.astype(jnp.float32)
    if (b_pad, h_pad) != (B, hidden_size):
        h0_p = jnp.zeros((b_pad, h_pad), jnp.float32).at[
            :B, :hidden_size].set(h0_p)

    body = functools.partial(_block_body, gk=in_pad, nt=nt, chunk=chunk)

    h_seq = pl.pallas_call(
        body,
        out_shape=jax.ShapeDtypeStruct((T, b_pad, h_pad), jnp.float32),
        grid=(T // nt,),
        in_specs=[
            pl.BlockSpec((nt, b_pad, in_pad), lambda i: (i, 0, 0)),
            pl.BlockSpec((b_pad, h_pad), lambda i: (0, 0)),
            pl.BlockSpec((k_pad, h_pad), lambda i: (0, 0)),
            pl.BlockSpec((1, h_pad), lambda i: (0, 0)),
        ],
        out_specs=pl.BlockSpec(memory_space=pl.ANY),
        scratch_shapes=[
            pltpu.VMEM((b_pad, k_pad), jnp.float32),
            pltpu.VMEM((2, chunk, b_pad, h_pad), jnp.float32),
            pltpu.SemaphoreType.DMA((2,)),
        ],
        compiler_params=pltpu.CompilerParams(
            dimension_semantics=("arbitrary",)),
    )(g_p, h0_p, w_cat.astype(jnp.float32), b_cat.astype(jnp.float32))

    if (b_pad, h_pad) != (B, hidden_size):
        h_seq = h_seq[:, :B, :hidden_size]
    return h_seq
